# baseline (speedup 1.0000x reference)
import functools

import jax
import jax.numpy as jnp
from jax.experimental import pallas as pl
from jax.experimental.pallas import tpu as pltpu

_TOPK = 5
_LANES = 128
_INF = float("inf")


def _knn_kernel(f_ref, mb_ref, m2_ref, out_ref, *L_refs, nk, bk, cw):
    j = pl.program_id(0)
    q = f_ref.shape[0]

    @pl.when(j == 0)
    def _init():
        for r in L_refs:
            r[...] = jnp.full((q, cw), _INF, jnp.float32)

    f = f_ref[...]
    mb = mb_ref[...]
    fm = jax.lax.dot_general(
        f, mb, (((1,), (1,)), ((), ())), preferred_element_type=jnp.float32
    )
    s = m2_ref[0, :][None, :] - 2.0 * fm

    L = [r[...] for r in L_refs]
    for c in range(bk // cw):
        v = s[:, c * cw:(c + 1) * cw]
        for t in range(_TOPK):
            lo = jnp.minimum(L[t], v)
            v = jnp.maximum(L[t], v)
            L[t] = lo
    for r, val in zip(L_refs, L):
        r[...] = val

    @pl.when(j == nk - 1)
    def _finish():
        f2 = jnp.sum(f * f, axis=1, keepdims=True)
        cand = jnp.concatenate(L, axis=1)
        w = cand.shape[1]
        lane = jax.lax.broadcasted_iota(jnp.int32, (q, w), 1)
        total = jnp.zeros((q, 1), jnp.float32)
        for _ in range(_TOPK):
            mn = jnp.min(cand, axis=1, keepdims=True)
            idx = jnp.min(jnp.where(cand == mn, lane, w), axis=1, keepdims=True)
            cand = jnp.where(lane == idx, _INF, cand)
            total = total + jnp.sqrt(jnp.maximum(f2 + mn, 1e-12))
        out_ref[...] = total


def _run(features, memory_bank, block_k, interpret=False):
    q, d = features.shape
    k_rows = memory_bank.shape[0]
    nk = -(-k_rows // block_k)
    kp = nk * block_k
    if kp != k_rows:
        memory_bank = jnp.pad(memory_bank, ((0, kp - k_rows), (0, 0)))
    m2 = jnp.sum(memory_bank * memory_bank, axis=1)
    if kp != k_rows:
        m2 = m2.at[k_rows:].set(_INF)
    m2 = m2.reshape(1, kp)

    cw = min(_LANES, block_k)
    assert block_k % cw == 0
    body = functools.partial(_knn_kernel, nk=nk, bk=block_k, cw=cw)
    out = pl.pallas_call(
        body,
        grid=(nk,),
        in_specs=[
            pl.BlockSpec((q, d), lambda j: (0, 0)),
            pl.BlockSpec((block_k, d), lambda j: (j, 0)),
            pl.BlockSpec((1, block_k), lambda j: (0, j)),
        ],
        out_specs=pl.BlockSpec((q, 1), lambda j: (0, 0)),
        out_shape=jax.ShapeDtypeStruct((q, 1), jnp.float32),
        scratch_shapes=[pltpu.VMEM((q, cw), jnp.float32) for _ in range(_TOPK)],
        interpret=interpret,
    )(features, memory_bank, m2)
    return out[:, 0]


def kernel(features, memory_bank, k):
    total = _run(features, memory_bank, block_k=2048)
    return total / k
